# baseline (device time: 60797 ns/iter reference)
import jax
import jax.numpy as jnp
from jax import lax
from jax.experimental import pallas as pl
from jax.experimental.pallas import tpu as pltpu

N_DEV = 4
SCALE = 0.08838834764831843


def _body(x_ref, wq_ref, wo_ref, k_hbm, v_hbm, out_ref,
          k_scr, v_scr, s1_buf, s2_buf, recv_buf,
          kv_sems, send_sems, recv_sems):
    my = lax.axis_index("i")
    partner_y = my - 2 * lax.rem(my, 2) + 1
    partner_x = 3 - my

    col0 = 256 * my
    k_dma = pltpu.make_async_copy(
        k_hbm.at[:, pl.ds(col0, 256)], k_scr, kv_sems.at[0])
    v_dma = pltpu.make_async_copy(
        v_hbm.at[:, pl.ds(col0, 256)], v_scr, kv_sems.at[1])
    k_dma.start()
    v_dma.start()

    barrier_sem = pltpu.get_barrier_semaphore()
    for nbr in (partner_y, partner_x):
        pl.semaphore_signal(
            barrier_sem, inc=1,
            device_id=(nbr,), device_id_type=pl.DeviceIdType.MESH,
        )
    pl.semaphore_wait(barrier_sem, 2)

    q_all = jnp.dot(x_ref[...], wq_ref[...],
                    preferred_element_type=jnp.float32) * SCALE
    k_dma.wait()
    v_dma.wait()

    heads = []
    for h in range(8):
        g = h // 4
        q = q_all[:, h * 128:(h + 1) * 128]
        k = k_scr[:, g * 128:(g + 1) * 128]
        s = lax.dot_general(
            q, k, (((1,), (1,)), ((), ())),
            preferred_element_type=jnp.float32,
        )
        m = jnp.max(s, axis=1, keepdims=True)
        p = jnp.exp(s - m)
        l = jnp.sum(p, axis=1, keepdims=True)
        o = jnp.dot(p, v_scr[:, g * 128:(g + 1) * 128],
                    preferred_element_type=jnp.float32) / l
        heads.append(o)
    attn = jnp.concatenate(heads, axis=1)

    partial = jnp.dot(attn, wo_ref[...], preferred_element_type=jnp.float32)

    s1_buf[0] = partial[:, :512]
    s1_buf[1] = partial[:, 512:]
    s1 = []
    for idx, tgt in ((0, partner_y), (1, partner_x)):
        rdma = pltpu.make_async_remote_copy(
            src_ref=s1_buf.at[idx], dst_ref=recv_buf.at[idx],
            send_sem=send_sems.at[idx], recv_sem=recv_sems.at[idx],
            device_id=(tgt,), device_id_type=pl.DeviceIdType.MESH,
        )
        rdma.start()
        s1.append(rdma)
    for rdma in s1:
        rdma.wait()

    s2_buf[0] = s1_buf[0] + recv_buf[0]
    s2_buf[1] = s1_buf[1] + recv_buf[1]
    s2 = []
    for idx, tgt in ((0, partner_x), (1, partner_y)):
        rdma = pltpu.make_async_remote_copy(
            src_ref=s2_buf.at[idx], dst_ref=recv_buf.at[idx + 2],
            send_sem=send_sems.at[idx + 2], recv_sem=recv_sems.at[idx + 2],
            device_id=(tgt,), device_id_type=pl.DeviceIdType.MESH,
        )
        rdma.start()
        s2.append(rdma)
    for rdma in s2:
        rdma.wait()

    out_ref[:, :512] = s2_buf[0] + recv_buf[2]
    out_ref[:, 512:] = s2_buf[1] + recv_buf[3]


def kernel(x, Wq, Wo, K_ext, V_ext):
    x2d = x[0]
    k2 = K_ext.reshape(4096, 1024)
    v2 = V_ext.reshape(4096, 1024)

    out = pl.pallas_call(
        _body,
        out_shape=jax.ShapeDtypeStruct((256, 1024), jnp.float32),
        in_specs=[pl.BlockSpec(memory_space=pltpu.VMEM)] * 3
        + [pl.BlockSpec(memory_space=pltpu.MemorySpace.HBM)] * 2,
        out_specs=pl.BlockSpec(memory_space=pltpu.VMEM),
        scratch_shapes=[
            pltpu.VMEM((4096, 256), jnp.float32),
            pltpu.VMEM((4096, 256), jnp.float32),
            pltpu.VMEM((2, 256, 512), jnp.float32),
            pltpu.VMEM((2, 256, 512), jnp.float32),
            pltpu.VMEM((4, 256, 512), jnp.float32),
            pltpu.SemaphoreType.DMA((2,)),
            pltpu.SemaphoreType.DMA((4,)),
            pltpu.SemaphoreType.DMA((4,)),
        ],
        compiler_params=pltpu.CompilerParams(collective_id=0),
    )(x2d, Wq, Wo, k2, v2)
    return out[None]


# device time: 39667 ns/iter; 1.5327x vs baseline; 1.5327x over previous
import jax
import jax.numpy as jnp
from jax import lax
from jax.experimental import pallas as pl
from jax.experimental.pallas import tpu as pltpu

N_DEV = 4
SCALE = 0.08838834764831843


def _body(x_ref, wq_ref, wo_ref, k_hbm, v_hbm, out_ref,
          k_scr, v_scr, s1_buf, s2_buf, recv_buf,
          kv_sems, send_sems, recv_sems):
    my = lax.axis_index("i")
    partner_y = my - 2 * lax.rem(my, 2) + 1
    partner_x = 3 - my

    kv_dmas = []
    for t, (hbm, scr) in enumerate(((k_hbm, k_scr), (v_hbm, v_scr))):
        for g in range(2):
            dma = pltpu.make_async_copy(
                hbm.at[0, :, 2 * my + g, :], scr.at[g],
                kv_sems.at[2 * t + g])
            dma.start()
            kv_dmas.append(dma)

    barrier_sem = pltpu.get_barrier_semaphore()
    for nbr in (partner_y, partner_x):
        pl.semaphore_signal(
            barrier_sem, inc=1,
            device_id=(nbr,), device_id_type=pl.DeviceIdType.MESH,
        )
    pl.semaphore_wait(barrier_sem, 2)

    q_all = jnp.dot(x_ref[...], wq_ref[...],
                    preferred_element_type=jnp.float32) * SCALE
    for dma in kv_dmas:
        dma.wait()

    heads = []
    for h in range(8):
        g = h // 4
        q = q_all[:, h * 128:(h + 1) * 128]
        s = lax.dot_general(
            q, k_scr[g], (((1,), (1,)), ((), ())),
            preferred_element_type=jnp.float32,
        )
        m = jnp.max(s, axis=1, keepdims=True)
        p = jnp.exp(s - m)
        l = jnp.sum(p, axis=1, keepdims=True)
        o = jnp.dot(p, v_scr[g], preferred_element_type=jnp.float32) / l
        heads.append(o)
    attn = jnp.concatenate(heads, axis=1)

    partial = jnp.dot(attn, wo_ref[...], preferred_element_type=jnp.float32)

    s1_buf[0] = partial[:, :512]
    s1_buf[1] = partial[:, 512:]
    s1 = []
    for idx, tgt in ((0, partner_y), (1, partner_x)):
        rdma = pltpu.make_async_remote_copy(
            src_ref=s1_buf.at[idx], dst_ref=recv_buf.at[idx],
            send_sem=send_sems.at[idx], recv_sem=recv_sems.at[idx],
            device_id=(tgt,), device_id_type=pl.DeviceIdType.MESH,
        )
        rdma.start()
        s1.append(rdma)
    for rdma in s1:
        rdma.wait()

    s2_buf[0] = s1_buf[0] + recv_buf[0]
    s2_buf[1] = s1_buf[1] + recv_buf[1]
    s2 = []
    for idx, tgt in ((0, partner_x), (1, partner_y)):
        rdma = pltpu.make_async_remote_copy(
            src_ref=s2_buf.at[idx], dst_ref=recv_buf.at[idx + 2],
            send_sem=send_sems.at[idx + 2], recv_sem=recv_sems.at[idx + 2],
            device_id=(tgt,), device_id_type=pl.DeviceIdType.MESH,
        )
        rdma.start()
        s2.append(rdma)
    for rdma in s2:
        rdma.wait()

    out_ref[:, :512] = s2_buf[0] + recv_buf[2]
    out_ref[:, 512:] = s2_buf[1] + recv_buf[3]


def kernel(x, Wq, Wo, K_ext, V_ext):
    x2d = x[0]

    out = pl.pallas_call(
        _body,
        out_shape=jax.ShapeDtypeStruct((256, 1024), jnp.float32),
        in_specs=[pl.BlockSpec(memory_space=pltpu.VMEM)] * 3
        + [pl.BlockSpec(memory_space=pltpu.MemorySpace.HBM)] * 2,
        out_specs=pl.BlockSpec(memory_space=pltpu.VMEM),
        scratch_shapes=[
            pltpu.VMEM((2, 4096, 128), jnp.float32),
            pltpu.VMEM((2, 4096, 128), jnp.float32),
            pltpu.VMEM((2, 256, 512), jnp.float32),
            pltpu.VMEM((2, 256, 512), jnp.float32),
            pltpu.VMEM((4, 256, 512), jnp.float32),
            pltpu.SemaphoreType.DMA((4,)),
            pltpu.SemaphoreType.DMA((4,)),
            pltpu.SemaphoreType.DMA((4,)),
        ],
        compiler_params=pltpu.CompilerParams(collective_id=0),
    )(x2d, Wq, Wo, K_ext, V_ext)
    return out[None]
